# baseline (speedup 1.0000x reference)
import jax
import jax.numpy as jnp
from jax.experimental import pallas as pl
from jax.experimental.pallas import tpu as pltpu


def _heads_kernel(x_ref, wc_ref, bc_ref, wr_ref, br_ref, wd_ref, bd_ref,
                  oc_ref, or_ref, od_ref):
    x = x_ref[0]
    dn = (((1,), (0,)), ((), ()))
    hi = jax.lax.Precision.HIGHEST
    oc_ref[0] = jax.lax.dot_general(
        wc_ref[...], x, dn, precision=hi,
        preferred_element_type=jnp.float32) + bc_ref[...]
    or_ref[0] = jax.lax.dot_general(
        wr_ref[...], x, dn, precision=hi,
        preferred_element_type=jnp.float32) + br_ref[...]
    od_ref[0] = jax.lax.dot_general(
        wd_ref[...], x, dn, precision=hi,
        preferred_element_type=jnp.float32) + bd_ref[...]


def kernel(x, W_cls, b_cls, W_reg, b_reg, W_dir, b_dir):
    B, C, H, W = x.shape
    Oc, Or, Od = W_cls.shape[0], W_reg.shape[0], W_dir.shape[0]
    N = H * W
    NB = 3456
    n_blocks = (N + NB - 1) // NB

    x2 = x.reshape(B, C, N)
    bc = b_cls[:, None]
    br = b_reg[:, None]
    bd = b_dir[:, None]

    full = lambda shape: pl.BlockSpec(shape, lambda b, n: (0, 0))
    outs = pl.pallas_call(
        _heads_kernel,
        grid=(B, n_blocks),
        in_specs=[
            pl.BlockSpec((1, C, NB), lambda b, n: (b, 0, n)),
            full((Oc, C)), full((Oc, 1)),
            full((Or, C)), full((Or, 1)),
            full((Od, C)), full((Od, 1)),
        ],
        out_specs=[
            pl.BlockSpec((1, Oc, NB), lambda b, n: (b, 0, n)),
            pl.BlockSpec((1, Or, NB), lambda b, n: (b, 0, n)),
            pl.BlockSpec((1, Od, NB), lambda b, n: (b, 0, n)),
        ],
        out_shape=[
            jax.ShapeDtypeStruct((B, Oc, N), jnp.float32),
            jax.ShapeDtypeStruct((B, Or, N), jnp.float32),
            jax.ShapeDtypeStruct((B, Od, N), jnp.float32),
        ],
        compiler_params=pltpu.CompilerParams(
            dimension_semantics=("parallel", "parallel")),
    )(x2, W_cls, bc, W_reg, br, W_dir, bd)

    cls_p, reg_p, dir_p = outs
    return (cls_p.reshape(B, Oc, H, W),
            reg_p.reshape(B, Or, H, W),
            dir_p.reshape(B, Od, H, W))


# speedup vs baseline: 1.6320x; 1.6320x over previous
import functools

import jax
import jax.numpy as jnp
from jax.experimental import pallas as pl
from jax.experimental.pallas import tpu as pltpu


def _heads_kernel(oc, od, w_ref, b_ref, x_ref, oc_ref, or_ref, od_ref):
    x = x_ref[0]
    dn = (((1,), (0,)), ((), ()))
    y = jax.lax.dot_general(
        w_ref[...], x, dn, preferred_element_type=jnp.float32) + b_ref[...]
    oc_ref[0] = y[:oc]
    or_ref[0] = y[oc:-od]
    od_ref[0] = y[-od:]


def kernel(x, W_cls, b_cls, W_reg, b_reg, W_dir, b_dir):
    B, C, H, W = x.shape
    Oc, Or, Od = W_cls.shape[0], W_reg.shape[0], W_dir.shape[0]
    N = H * W
    NB = 3456
    n_blocks = (N + NB - 1) // NB

    x2 = x.reshape(B, C, N)
    Ot = Oc + Or + Od
    w_all = jnp.concatenate([W_cls, W_reg, W_dir], axis=0)
    b_all = jnp.concatenate([b_cls, b_reg, b_dir], axis=0)[:, None]

    body = functools.partial(_heads_kernel, Oc, Od)
    full = lambda shape: pl.BlockSpec(shape, lambda b, n: (0, 0))
    outs = pl.pallas_call(
        body,
        grid=(B, n_blocks),
        in_specs=[
            full((Ot, C)), full((Ot, 1)),
            pl.BlockSpec((1, C, NB), lambda b, n: (b, 0, n)),
        ],
        out_specs=[
            pl.BlockSpec((1, Oc, NB), lambda b, n: (b, 0, n)),
            pl.BlockSpec((1, Or, NB), lambda b, n: (b, 0, n)),
            pl.BlockSpec((1, Od, NB), lambda b, n: (b, 0, n)),
        ],
        out_shape=[
            jax.ShapeDtypeStruct((B, Oc, N), jnp.float32),
            jax.ShapeDtypeStruct((B, Or, N), jnp.float32),
            jax.ShapeDtypeStruct((B, Od, N), jnp.float32),
        ],
        compiler_params=pltpu.CompilerParams(
            dimension_semantics=("parallel", "parallel")),
    )(w_all, b_all, x2)

    cls_p, reg_p, dir_p = outs
    return (cls_p.reshape(B, Oc, H, W),
            reg_p.reshape(B, Or, H, W),
            dir_p.reshape(B, Od, H, W))
